# baseline (speedup 1.0000x reference)
import functools

import jax
import jax.numpy as jnp
from jax import lax
from jax.experimental import pallas as pl
from jax.experimental.pallas import tpu as pltpu
from jax.experimental.pallas import tpu_sc as plsc

_BATCH = 4096
_EMB = 16
_NU = 2
_NI = 10
_LAYER = 256
_XDIM = 15448
_TOT = _NU * _EMB + _NI * _EMB + _XDIM

_NC = 2
_NS = 16
_NW = _NC * _NS

_CHUNK = 128

_UB = _BATCH * _NU
_IB = _BATCH * _NI
_U_PER = _UB // _NW
_I_PER = _IB // _NW
_UC = _U_PER // _CHUNK
_IC = _I_PER // _CHUNK


def _sc_gather_body(u_idx, i_idx, u_tab, i_tab, u_out, i_out,
                    uidx_v, urows_v, iidx_v, irows_v, sem):
    wid = lax.axis_index("s") * _NC + lax.axis_index("c")
    pltpu.sync_copy(u_idx.at[pl.ds(wid * _U_PER, _U_PER)], uidx_v)
    pltpu.sync_copy(i_idx.at[pl.ds(wid * _I_PER, _I_PER)], iidx_v)
    cps = []
    for j in range(_UC):
        cps.append(pltpu.async_copy(
            u_tab.at[uidx_v.at[pl.ds(j * _CHUNK, _CHUNK)]],
            urows_v.at[pl.ds(j * _CHUNK, _CHUNK)], sem))
    for j in range(_IC):
        cps.append(pltpu.async_copy(
            i_tab.at[iidx_v.at[pl.ds(j * _CHUNK, _CHUNK)]],
            irows_v.at[pl.ds(j * _CHUNK, _CHUNK)], sem))
    for cp in cps:
        cp.wait()
    pltpu.sync_copy(urows_v, u_out.at[pl.ds(wid * _U_PER, _U_PER)])
    pltpu.sync_copy(irows_v, i_out.at[pl.ds(wid * _I_PER, _I_PER)])


@functools.lru_cache(maxsize=1)
def _sc_gather():
    return pl.kernel(
        _sc_gather_body,
        mesh=plsc.VectorSubcoreMesh(core_axis_name="c", subcore_axis_name="s"),
        out_type=[
            jax.ShapeDtypeStruct((_UB, _EMB), jnp.float32),
            jax.ShapeDtypeStruct((_IB, _EMB), jnp.float32),
        ],
        scratch_types=[
            pltpu.VMEM((_U_PER,), jnp.int32),
            pltpu.VMEM((_U_PER, _EMB), jnp.float32),
            pltpu.VMEM((_I_PER,), jnp.int32),
            pltpu.VMEM((_I_PER, _EMB), jnp.float32),
            pltpu.SemaphoreType.DMA,
        ],
        compiler_params=pltpu.CompilerParams(use_tc_tiling_on_sc=False),
    )


_TM = 256
_MT = _BATCH // _TM


def _mlp_body(x_ref, ue_ref, ie_ref, w1_ref, g1_ref, be1_ref,
              w2_ref, g2_ref, be2_ref, w3_ref, b3_ref, out_ref, h1_scr):
    i = pl.program_id(0)
    xw = jnp.dot(x_ref[...], w1_ref[_NU * _EMB + _NI * _EMB:, :],
                 preferred_element_type=jnp.float32)
    h1_scr[pl.ds(i * _TM, _TM), :] = xw

    @pl.when(i == _MT - 1)
    def _():
        emb_h = (
            jnp.dot(ue_ref[...], w1_ref[: _NU * _EMB, :],
                    preferred_element_type=jnp.float32)
            + jnp.dot(ie_ref[...], w1_ref[_NU * _EMB:_NU * _EMB + _NI * _EMB, :],
                      preferred_element_type=jnp.float32))
        h1 = h1_scr[...] + emb_h
        m1 = jnp.mean(h1, axis=0, keepdims=True)
        v1 = jnp.mean((h1 - m1) * (h1 - m1), axis=0, keepdims=True)
        h = (h1 - m1) * lax.rsqrt(v1 + 1e-5) * g1_ref[...] + be1_ref[...]
        h = jnp.maximum(h, 0.0)
        h2 = jnp.dot(h, w2_ref[...], preferred_element_type=jnp.float32)
        m2 = jnp.mean(h2, axis=0, keepdims=True)
        v2 = jnp.mean((h2 - m2) * (h2 - m2), axis=0, keepdims=True)
        h2 = (h2 - m2) * lax.rsqrt(v2 + 1e-5) * g2_ref[...] + be2_ref[...]
        h2 = jnp.maximum(h2, 0.0)
        out_ref[...] = (jnp.dot(h2, w3_ref[...],
                                preferred_element_type=jnp.float32)
                        + b3_ref[...])


def _mlp(x, ue, ie, W1, g1, be1, W2, g2, be2, W3, b3):
    nde = _NU * _EMB + _NI * _EMB
    return pl.pallas_call(
        _mlp_body,
        grid=(_MT,),
        in_specs=[
            pl.BlockSpec((_TM, _XDIM), lambda i: (i, 0)),
            pl.BlockSpec((_BATCH, _NU * _EMB), lambda i: (0, 0)),
            pl.BlockSpec((_BATCH, _NI * _EMB), lambda i: (0, 0)),
            pl.BlockSpec((_TOT, _LAYER), lambda i: (0, 0)),
            pl.BlockSpec((1, _LAYER), lambda i: (0, 0)),
            pl.BlockSpec((1, _LAYER), lambda i: (0, 0)),
            pl.BlockSpec((_LAYER, _LAYER), lambda i: (0, 0)),
            pl.BlockSpec((1, _LAYER), lambda i: (0, 0)),
            pl.BlockSpec((1, _LAYER), lambda i: (0, 0)),
            pl.BlockSpec((_LAYER, _NI), lambda i: (0, 0)),
            pl.BlockSpec((1, _NI), lambda i: (0, 0)),
        ],
        out_specs=pl.BlockSpec((_BATCH, _NI), lambda i: (0, 0)),
        out_shape=jax.ShapeDtypeStruct((_BATCH, _NI), jnp.float32),
        scratch_shapes=[pltpu.VMEM((_BATCH, _LAYER), jnp.float32)],
        compiler_params=pltpu.CompilerParams(
            vmem_limit_bytes=100 * 1024 * 1024),
    )(x, ue, ie, W1, g1, be1, W2, g2, be2, W3, b3)


def kernel(x, u_cats, i_cats, user_table, item_table,
           W1, b1, g1, be1, W2, b2, g2, be2, W3, b3):
    u_idx = u_cats.reshape(_UB)
    i_idx = i_cats.reshape(_IB)
    u_rows, i_rows = _sc_gather()(u_idx, i_idx, user_table, item_table)
    ue = u_rows.reshape(_BATCH, _NU * _EMB)
    ie = i_rows.reshape(_BATCH, _NI * _EMB)
    return _mlp(x, ue, ie, W1,
                g1.reshape(1, _LAYER), be1.reshape(1, _LAYER),
                W2, g2.reshape(1, _LAYER), be2.reshape(1, _LAYER),
                W3, b3.reshape(1, _NI))


# speedup vs baseline: 1.2365x; 1.2365x over previous
import functools

import jax
import jax.numpy as jnp
from jax import lax
from jax.experimental import pallas as pl
from jax.experimental.pallas import tpu as pltpu
from jax.experimental.pallas import tpu_sc as plsc

_BATCH = 4096
_EMB = 16
_NU = 2
_NI = 10
_LAYER = 256
_XDIM = 15448
_TOT = _NU * _EMB + _NI * _EMB + _XDIM

_NC = 2
_NS = 16
_NW = _NC * _NS

_CHUNK = 128

_UB = _BATCH * _NU
_IB = _BATCH * _NI
_U_PER = _UB // _NW
_I_PER = _IB // _NW
_UC = _U_PER // _CHUNK
_IC = _I_PER // _CHUNK


def _sc_gather_body(u_idx, i_idx, u_tab, i_tab, u_out, i_out,
                    uidx_v, urows_v, iidx_v, irows_v, sem):
    wid = lax.axis_index("s") * _NC + lax.axis_index("c")
    pltpu.sync_copy(u_idx.at[pl.ds(wid * _U_PER, _U_PER)], uidx_v)
    pltpu.sync_copy(i_idx.at[pl.ds(wid * _I_PER, _I_PER)], iidx_v)
    cps = []
    for j in range(_UC):
        cps.append(pltpu.async_copy(
            u_tab.at[uidx_v.at[pl.ds(j * _CHUNK, _CHUNK)]],
            urows_v.at[pl.ds(j * _CHUNK, _CHUNK)], sem))
    for j in range(_IC):
        cps.append(pltpu.async_copy(
            i_tab.at[iidx_v.at[pl.ds(j * _CHUNK, _CHUNK)]],
            irows_v.at[pl.ds(j * _CHUNK, _CHUNK)], sem))
    for cp in cps:
        cp.wait()
    pltpu.sync_copy(urows_v, u_out.at[pl.ds(wid * _U_PER, _U_PER)])
    pltpu.sync_copy(irows_v, i_out.at[pl.ds(wid * _I_PER, _I_PER)])


@functools.lru_cache(maxsize=1)
def _sc_gather():
    return pl.kernel(
        _sc_gather_body,
        mesh=plsc.VectorSubcoreMesh(core_axis_name="c", subcore_axis_name="s"),
        out_type=[
            jax.ShapeDtypeStruct((_UB, _EMB), jnp.float32),
            jax.ShapeDtypeStruct((_IB, _EMB), jnp.float32),
        ],
        scratch_types=[
            pltpu.VMEM((_U_PER,), jnp.int32),
            pltpu.VMEM((_U_PER, _EMB), jnp.float32),
            pltpu.VMEM((_I_PER,), jnp.int32),
            pltpu.VMEM((_I_PER, _EMB), jnp.float32),
            pltpu.SemaphoreType.DMA,
        ],
        compiler_params=pltpu.CompilerParams(use_tc_tiling_on_sc=False),
    )


_TM = 256
_MT = _BATCH // _TM


def _mlp_body(xt_ref, ue_ref, ie_ref, w1_ref, g1_ref, be1_ref,
              w2_ref, g2_ref, be2_ref, w3_ref, b3_ref, out_ref, h1_scr):
    i = pl.program_id(0)
    xw = lax.dot_general(
        xt_ref[...], w1_ref[_NU * _EMB + _NI * _EMB:, :],
        dimension_numbers=(((0,), (0,)), ((), ())),
        preferred_element_type=jnp.float32)
    h1_scr[pl.ds(i * _TM, _TM), :] = xw

    @pl.when(i == _MT - 1)
    def _():
        emb_h = (
            jnp.dot(ue_ref[...], w1_ref[: _NU * _EMB, :],
                    preferred_element_type=jnp.float32)
            + jnp.dot(ie_ref[...], w1_ref[_NU * _EMB:_NU * _EMB + _NI * _EMB, :],
                      preferred_element_type=jnp.float32))
        h1 = h1_scr[...] + emb_h
        m1 = jnp.mean(h1, axis=0, keepdims=True)
        v1 = jnp.mean((h1 - m1) * (h1 - m1), axis=0, keepdims=True)
        h = (h1 - m1) * lax.rsqrt(v1 + 1e-5) * g1_ref[...] + be1_ref[...]
        h = jnp.maximum(h, 0.0)
        h2 = jnp.dot(h, w2_ref[...], preferred_element_type=jnp.float32)
        m2 = jnp.mean(h2, axis=0, keepdims=True)
        v2 = jnp.mean((h2 - m2) * (h2 - m2), axis=0, keepdims=True)
        h2 = (h2 - m2) * lax.rsqrt(v2 + 1e-5) * g2_ref[...] + be2_ref[...]
        h2 = jnp.maximum(h2, 0.0)
        out_ref[...] = (jnp.dot(h2, w3_ref[...],
                                preferred_element_type=jnp.float32)
                        + b3_ref[...])


def _mlp(xt, ue, ie, W1, g1, be1, W2, g2, be2, W3, b3):
    return pl.pallas_call(
        _mlp_body,
        grid=(_MT,),
        in_specs=[
            pl.BlockSpec((_XDIM, _TM), lambda i: (0, i)),
            pl.BlockSpec((_BATCH, _NU * _EMB), lambda i: (0, 0)),
            pl.BlockSpec((_BATCH, _NI * _EMB), lambda i: (0, 0)),
            pl.BlockSpec((_TOT, _LAYER), lambda i: (0, 0)),
            pl.BlockSpec((1, _LAYER), lambda i: (0, 0)),
            pl.BlockSpec((1, _LAYER), lambda i: (0, 0)),
            pl.BlockSpec((_LAYER, _LAYER), lambda i: (0, 0)),
            pl.BlockSpec((1, _LAYER), lambda i: (0, 0)),
            pl.BlockSpec((1, _LAYER), lambda i: (0, 0)),
            pl.BlockSpec((_LAYER, _NI), lambda i: (0, 0)),
            pl.BlockSpec((1, _NI), lambda i: (0, 0)),
        ],
        out_specs=pl.BlockSpec((_BATCH, _NI), lambda i: (0, 0)),
        out_shape=jax.ShapeDtypeStruct((_BATCH, _NI), jnp.float32),
        scratch_shapes=[pltpu.VMEM((_BATCH, _LAYER), jnp.float32)],
        compiler_params=pltpu.CompilerParams(
            vmem_limit_bytes=100 * 1024 * 1024),
    )(xt, ue, ie, W1, g1, be1, W2, g2, be2, W3, b3)


def kernel(x, u_cats, i_cats, user_table, item_table,
           W1, b1, g1, be1, W2, b2, g2, be2, W3, b3):
    u_idx = u_cats.reshape(_UB)
    i_idx = i_cats.reshape(_IB)
    u_rows, i_rows = _sc_gather()(u_idx, i_idx, user_table, item_table)
    ue = u_rows.reshape(_BATCH, _NU * _EMB)
    ie = i_rows.reshape(_BATCH, _NI * _EMB)
    return _mlp(x.T, ue, ie, W1,
                g1.reshape(1, _LAYER), be1.reshape(1, _LAYER),
                W2, g2.reshape(1, _LAYER), be2.reshape(1, _LAYER),
                W3, b3.reshape(1, _NI))
